# baseline (speedup 1.0000x reference)
import jax
import jax.numpy as jnp
from jax.experimental import pallas as pl
from jax.experimental.pallas import tpu as pltpu

_BS = 256


def _gelu_exact(x):
    return x * 0.5 * (1.0 + jax.lax.erf(x * 0.7071067811865476))


def _fuse_body(sref, prompt_ref, ida_ref, idb_ref, mask_ref, maskt_ref,
               w11_ref, w21_ref, w12_ref, w22_ref,
               vec_ref, ln1_ref, out_ref,
               w11a_s, w11b_s, w21_s, w12_s, w22_s):
    i = pl.program_id(0)
    bs = _BS
    D = prompt_ref.shape[1]

    @pl.when(i == 0)
    def _cast_weights():
        w11a_s[...] = w11_ref[:, :D].astype(jnp.bfloat16)
        w11b_s[...] = w11_ref[:, D:].astype(jnp.bfloat16)
        w21_s[...] = w21_ref[...].astype(jnp.bfloat16)
        w12_s[...] = w12_ref[...].astype(jnp.bfloat16)
        w22_s[...] = w22_ref[...].astype(jnp.bfloat16)

    a = prompt_ref[...]
    m = mask_ref[0]
    tri = (jax.lax.broadcasted_iota(jnp.int32, (bs, bs), 0)
           <= jax.lax.broadcasted_iota(jnp.int32, (bs, bs), 1))
    cs_f = jnp.dot(m.astype(jnp.bfloat16), tri.astype(jnp.bfloat16),
                   preferred_element_type=jnp.float32)
    cs = cs_f.astype(jnp.int32)
    excl_i = sref[0, i]
    b0_i = sref[1, i]
    off = excl_i + cs - 1 - b0_i * bs
    off = jnp.clip(off, 0, 2 * bs - 1)

    iota = jax.lax.broadcasted_iota(jnp.int32, (2 * bs, bs), 0)
    ohT = (iota == off).astype(jnp.bfloat16)

    dimn = (((0,), (0,)), ((), ()))
    ida = ida_ref[0, :, 0, :].astype(jnp.bfloat16)
    idb = idb_ref[0, :, 0, :].astype(jnp.bfloat16)
    p = jax.lax.dot_general(ohT[:bs], ida, dimn,
                            preferred_element_type=jnp.float32)
    p = p + jax.lax.dot_general(ohT[bs:], idb, dimn,
                                preferred_element_type=jnp.float32)

    inv = 1.0 / (2 * D)
    mu = (jnp.sum(a, axis=1, keepdims=True)
          + jnp.sum(p, axis=1, keepdims=True)) * inv
    sq = (jnp.sum(a * a, axis=1, keepdims=True)
          + jnp.sum(p * p, axis=1, keepdims=True)) * inv
    rstd = jax.lax.rsqrt(sq - mu * mu + 1e-5)
    na = ((a - mu) * rstd) * ln1_ref[0:1, :D] + ln1_ref[1:2, :D]
    npair = ((p - mu) * rstd) * ln1_ref[0:1, D:] + ln1_ref[1:2, D:]

    def mm(x, w_s):
        return jax.lax.dot_general(
            x.astype(jnp.bfloat16), w_s[...], (((1,), (1,)), ((), ())),
            preferred_element_type=jnp.float32)

    h = mm(na, w11a_s) + mm(npair, w11b_s) + vec_ref[0:1, :]
    h = _gelu_exact(h)
    h = mm(h, w21_s) + vec_ref[1:2, :]
    x1 = h + a

    mu2 = jnp.mean(x1, axis=1, keepdims=True)
    sq2 = jnp.mean(x1 * x1, axis=1, keepdims=True)
    rstd2 = jax.lax.rsqrt(sq2 - mu2 * mu2 + 1e-5)
    n2 = ((x1 - mu2) * rstd2) * vec_ref[2:3, :] + vec_ref[3:4, :]
    h = mm(n2, w12_s) + vec_ref[4:5, :]
    h = _gelu_exact(h)
    h = mm(h, w22_s) + vec_ref[5:6, :]
    x2 = h + x1

    muf = jnp.mean(x2, axis=1, keepdims=True)
    sqf = jnp.mean(x2 * x2, axis=1, keepdims=True)
    rstdf = jax.lax.rsqrt(sqf - muf * muf + 1e-5)
    y = ((x2 - muf) * rstdf) * vec_ref[6:7, :] + vec_ref[7:8, :]

    mf = maskt_ref[0]
    out_ref[...] = y * mf + a * (1.0 - mf)


def kernel(prompt_embeds, id_embeds, class_tokens_mask, ln1_g, ln1_b,
           w1_1, b1_1, w2_1, b2_1, ln2_g, ln2_b, w1_2, b1_2, w2_2, b2_2,
           lnf_g, lnf_b, *, interpret=False):
    B, S, D = prompt_embeds.shape
    bs = _BS
    nb = S // bs
    flat_prompt = prompt_embeds.reshape(S, D)
    M = id_embeds.shape[1]
    nid_b = M // bs

    mask = class_tokens_mask.reshape(S).astype(jnp.int32)
    mask3 = mask.reshape(nb, 1, bs)
    maskt = mask.reshape(nb, bs, 1).astype(jnp.float32)
    counts = jnp.sum(mask.reshape(nb, bs), axis=1)
    excl = jnp.concatenate([jnp.zeros((1,), jnp.int32),
                            jnp.cumsum(counts)[:-1].astype(jnp.int32)])
    b0 = jnp.minimum(excl // bs, nid_b - 1)
    b1 = jnp.minimum(b0 + 1, nid_b - 1)
    scal = jnp.stack([excl, b0, b1]).astype(jnp.int32)

    vec = jnp.stack([b1_1, b2_1, ln2_g, ln2_b, b1_2, b2_2, lnf_g, lnf_b])
    ln1 = jnp.stack([ln1_g, ln1_b])

    bf = jnp.bfloat16
    grid_spec = pltpu.PrefetchScalarGridSpec(
        num_scalar_prefetch=1,
        grid=(nb,),
        in_specs=[
            pl.BlockSpec((bs, D), lambda i, s: (i, 0)),
            pl.BlockSpec((1, bs, 1, D), lambda i, s: (0, s[1, i], 0, 0)),
            pl.BlockSpec((1, bs, 1, D), lambda i, s: (0, s[2, i], 0, 0)),
            pl.BlockSpec((1, 1, bs), lambda i, s: (i, 0, 0)),
            pl.BlockSpec((1, bs, 1), lambda i, s: (i, 0, 0)),
            pl.BlockSpec((D, 2 * D), lambda i, s: (0, 0)),
            pl.BlockSpec((D, D), lambda i, s: (0, 0)),
            pl.BlockSpec((D, D), lambda i, s: (0, 0)),
            pl.BlockSpec((D, D), lambda i, s: (0, 0)),
            pl.BlockSpec((8, D), lambda i, s: (0, 0)),
            pl.BlockSpec((2, 2 * D), lambda i, s: (0, 0)),
        ],
        out_specs=pl.BlockSpec((bs, D), lambda i, s: (i, 0)),
        scratch_shapes=[
            pltpu.VMEM((D, D), bf), pltpu.VMEM((D, D), bf),
            pltpu.VMEM((D, D), bf), pltpu.VMEM((D, D), bf),
            pltpu.VMEM((D, D), bf),
        ],
    )
    out = pl.pallas_call(
        _fuse_body,
        grid_spec=grid_spec,
        out_shape=jax.ShapeDtypeStruct((S, D), jnp.float32),
        compiler_params=pltpu.CompilerParams(
            dimension_semantics=("arbitrary",),
            vmem_limit_bytes=100 * 1024 * 1024),
        interpret=interpret,
    )(scal, flat_prompt, id_embeds, id_embeds, mask3, maskt,
      w1_1, w2_1, w1_2, w2_2, vec, ln1)
    return out.reshape(B, S, D)


# speedup vs baseline: 1.1569x; 1.1569x over previous
import jax
import jax.numpy as jnp
from jax.experimental import pallas as pl
from jax.experimental.pallas import tpu as pltpu

_BS = 256


def _gelu_exact(x):
    return x * 0.5 * (1.0 + jax.lax.erf(x * 0.7071067811865476))


def _fuse_body(sref, prompt_ref, ida_ref, idb_ref, mask_ref, maskt_ref,
               w11_ref, w21_ref, w12_ref, w22_ref,
               vec_ref, ln1_ref, out_ref,
               w11a_s, w11b_s, w21_s, w12_s, w22_s):
    i = pl.program_id(0)
    bs = _BS
    D = prompt_ref.shape[1]

    @pl.when(i == 0)
    def _cast_weights():
        w11a_s[...] = w11_ref[:, :D].astype(jnp.bfloat16)
        w11b_s[...] = w11_ref[:, D:].astype(jnp.bfloat16)
        w21_s[...] = w21_ref[...].astype(jnp.bfloat16)
        w12_s[...] = w12_ref[...].astype(jnp.bfloat16)
        w22_s[...] = w22_ref[...].astype(jnp.bfloat16)

    a = prompt_ref[...]
    m = mask_ref[0]
    tri = (jax.lax.broadcasted_iota(jnp.int32, (bs, bs), 0)
           <= jax.lax.broadcasted_iota(jnp.int32, (bs, bs), 1))
    cs_f = jnp.dot(m.astype(jnp.bfloat16), tri.astype(jnp.bfloat16),
                   preferred_element_type=jnp.float32)
    cs = cs_f.astype(jnp.int32)
    excl_i = sref[0, i]
    b0_i = sref[1, i]
    off = excl_i + cs - 1 - b0_i * bs
    off = jnp.clip(off, 0, 2 * bs - 1)

    iota = jax.lax.broadcasted_iota(jnp.int32, (2 * bs, bs), 0)
    ohT = (iota == off).astype(jnp.bfloat16)

    dimn = (((0,), (0,)), ((), ()))
    p = jax.lax.dot_general(ohT[:bs], ida_ref[...], dimn,
                            preferred_element_type=jnp.float32)
    p = p + jax.lax.dot_general(ohT[bs:], idb_ref[...], dimn,
                                preferred_element_type=jnp.float32)

    inv = 1.0 / (2 * D)
    mu = (jnp.sum(a, axis=1, keepdims=True)
          + jnp.sum(p, axis=1, keepdims=True)) * inv
    sq = (jnp.sum(a * a, axis=1, keepdims=True)
          + jnp.sum(p * p, axis=1, keepdims=True)) * inv
    rstd = jax.lax.rsqrt(sq - mu * mu + 1e-5)
    na = ((a - mu) * rstd) * ln1_ref[0:1, :D] + ln1_ref[1:2, :D]
    npair = ((p - mu) * rstd) * ln1_ref[0:1, D:] + ln1_ref[1:2, D:]

    def mm(x, w_s):
        return jax.lax.dot_general(
            x.astype(jnp.bfloat16), w_s[...], (((1,), (1,)), ((), ())),
            preferred_element_type=jnp.float32)

    h = mm(na, w11a_s) + mm(npair, w11b_s) + vec_ref[0:1, :]
    h = _gelu_exact(h)
    h = mm(h, w21_s) + vec_ref[1:2, :]
    x1 = h + a

    mu2 = jnp.mean(x1, axis=1, keepdims=True)
    sq2 = jnp.mean(x1 * x1, axis=1, keepdims=True)
    rstd2 = jax.lax.rsqrt(sq2 - mu2 * mu2 + 1e-5)
    n2 = ((x1 - mu2) * rstd2) * vec_ref[2:3, :] + vec_ref[3:4, :]
    h = mm(n2, w12_s) + vec_ref[4:5, :]
    h = _gelu_exact(h)
    h = mm(h, w22_s) + vec_ref[5:6, :]
    x2 = h + x1

    muf = jnp.mean(x2, axis=1, keepdims=True)
    sqf = jnp.mean(x2 * x2, axis=1, keepdims=True)
    rstdf = jax.lax.rsqrt(sqf - muf * muf + 1e-5)
    y = ((x2 - muf) * rstdf) * vec_ref[6:7, :] + vec_ref[7:8, :]

    mf = maskt_ref[0]
    out_ref[...] = y * mf + a * (1.0 - mf)


def kernel(prompt_embeds, id_embeds, class_tokens_mask, ln1_g, ln1_b,
           w1_1, b1_1, w2_1, b2_1, ln2_g, ln2_b, w1_2, b1_2, w2_2, b2_2,
           lnf_g, lnf_b, *, interpret=False):
    B, S, D = prompt_embeds.shape
    bs = _BS
    nb = S // bs
    flat_prompt = prompt_embeds.reshape(S, D)
    flat_id = id_embeds.reshape(-1, D).astype(jnp.bfloat16)
    M = flat_id.shape[0]
    nid_b = M // bs

    mask = class_tokens_mask.reshape(S).astype(jnp.int32)
    mask3 = mask.reshape(nb, 1, bs)
    maskt = mask.reshape(nb, bs, 1).astype(jnp.float32)
    counts = jnp.sum(mask.reshape(nb, bs), axis=1)
    excl = jnp.concatenate([jnp.zeros((1,), jnp.int32),
                            jnp.cumsum(counts)[:-1].astype(jnp.int32)])
    b0 = jnp.minimum(excl // bs, nid_b - 1)
    b1 = jnp.minimum(b0 + 1, nid_b - 1)
    scal = jnp.stack([excl, b0, b1]).astype(jnp.int32)

    vec = jnp.stack([b1_1, b2_1, ln2_g, ln2_b, b1_2, b2_2, lnf_g, lnf_b])
    ln1 = jnp.stack([ln1_g, ln1_b])

    bf = jnp.bfloat16
    grid_spec = pltpu.PrefetchScalarGridSpec(
        num_scalar_prefetch=1,
        grid=(nb,),
        in_specs=[
            pl.BlockSpec((bs, D), lambda i, s: (i, 0)),
            pl.BlockSpec((bs, D), lambda i, s: (s[1, i], 0)),
            pl.BlockSpec((bs, D), lambda i, s: (s[2, i], 0)),
            pl.BlockSpec((1, 1, bs), lambda i, s: (i, 0, 0)),
            pl.BlockSpec((1, bs, 1), lambda i, s: (i, 0, 0)),
            pl.BlockSpec((D, 2 * D), lambda i, s: (0, 0)),
            pl.BlockSpec((D, D), lambda i, s: (0, 0)),
            pl.BlockSpec((D, D), lambda i, s: (0, 0)),
            pl.BlockSpec((D, D), lambda i, s: (0, 0)),
            pl.BlockSpec((8, D), lambda i, s: (0, 0)),
            pl.BlockSpec((2, 2 * D), lambda i, s: (0, 0)),
        ],
        out_specs=pl.BlockSpec((bs, D), lambda i, s: (i, 0)),
        scratch_shapes=[
            pltpu.VMEM((D, D), bf), pltpu.VMEM((D, D), bf),
            pltpu.VMEM((D, D), bf), pltpu.VMEM((D, D), bf),
            pltpu.VMEM((D, D), bf),
        ],
    )
    out = pl.pallas_call(
        _fuse_body,
        grid_spec=grid_spec,
        out_shape=jax.ShapeDtypeStruct((S, D), jnp.float32),
        compiler_params=pltpu.CompilerParams(
            dimension_semantics=("arbitrary",),
            vmem_limit_bytes=100 * 1024 * 1024),
        interpret=interpret,
    )(scal, flat_prompt, flat_id, flat_id, mask3, maskt,
      w1_1, w2_1, w1_2, w2_2, vec, ln1)
    return out.reshape(B, S, D)


# speedup vs baseline: 1.2071x; 1.0434x over previous
import jax
import jax.numpy as jnp
from jax.experimental import pallas as pl
from jax.experimental.pallas import tpu as pltpu

_BS = 256


def _gelu_exact(x):
    return x * 0.5 * (1.0 + jax.lax.erf(x * 0.7071067811865476))


def _fuse_body(sref, prompt_ref, ida_ref, idb_ref, mask_ref,
               w11_ref, w21_ref, w12_ref, w22_ref,
               vec_ref, ln1_ref, out_ref,
               w11a_s, w11b_s, w21_s, w12_s, w22_s):
    i = pl.program_id(0)
    bs = _BS
    D = prompt_ref.shape[1]

    @pl.when(i == 0)
    def _cast_weights():
        w11a_s[...] = w11_ref[:, :D].astype(jnp.bfloat16)
        w11b_s[...] = w11_ref[:, D:].astype(jnp.bfloat16)
        w21_s[...] = w21_ref[...].astype(jnp.bfloat16)
        w12_s[...] = w12_ref[...].astype(jnp.bfloat16)
        w22_s[...] = w22_ref[...].astype(jnp.bfloat16)

    a = prompt_ref[...]
    m = mask_ref[0]
    tri = (jax.lax.broadcasted_iota(jnp.int32, (bs, bs), 0)
           <= jax.lax.broadcasted_iota(jnp.int32, (bs, bs), 1))
    cs_f = jnp.dot(m.astype(jnp.bfloat16), tri.astype(jnp.bfloat16),
                   preferred_element_type=jnp.float32)
    cs = cs_f.astype(jnp.int32)
    excl_i = sref[0, i]
    b0_i = sref[1, i]
    off = excl_i + cs - 1 - b0_i * bs
    off = jnp.clip(off, 0, 2 * bs - 1)

    iota = jax.lax.broadcasted_iota(jnp.int32, (2 * bs, bs), 0)
    ohT = (iota == off).astype(jnp.bfloat16)

    dimn = (((0,), (0,)), ((), ()))
    p = jax.lax.dot_general(ohT[:bs], ida_ref[...], dimn,
                            preferred_element_type=jnp.float32)
    p = p + jax.lax.dot_general(ohT[bs:], idb_ref[...], dimn,
                                preferred_element_type=jnp.float32)

    inv = 1.0 / (2 * D)
    mu = (jnp.sum(a, axis=1, keepdims=True)
          + jnp.sum(p, axis=1, keepdims=True)) * inv
    sq = (jnp.sum(a * a, axis=1, keepdims=True)
          + jnp.sum(p * p, axis=1, keepdims=True)) * inv
    rstd = jax.lax.rsqrt(sq - mu * mu + 1e-5)
    na = ((a - mu) * rstd) * ln1_ref[0:1, :D] + ln1_ref[1:2, :D]
    npair = ((p - mu) * rstd) * ln1_ref[0:1, D:] + ln1_ref[1:2, D:]

    def mm(x, w_s):
        return jax.lax.dot_general(
            x.astype(jnp.bfloat16), w_s[...], (((1,), (1,)), ((), ())),
            preferred_element_type=jnp.float32)

    h = mm(na, w11a_s) + mm(npair, w11b_s) + vec_ref[0:1, :]
    h = _gelu_exact(h)
    h = mm(h, w21_s) + vec_ref[1:2, :]
    x1 = h + a

    mu2 = jnp.mean(x1, axis=1, keepdims=True)
    sq2 = jnp.mean(x1 * x1, axis=1, keepdims=True)
    rstd2 = jax.lax.rsqrt(sq2 - mu2 * mu2 + 1e-5)
    n2 = ((x1 - mu2) * rstd2) * vec_ref[2:3, :] + vec_ref[3:4, :]
    h = mm(n2, w12_s) + vec_ref[4:5, :]
    h = _gelu_exact(h)
    h = mm(h, w22_s) + vec_ref[5:6, :]
    x2 = h + x1

    muf = jnp.mean(x2, axis=1, keepdims=True)
    sqf = jnp.mean(x2 * x2, axis=1, keepdims=True)
    rstdf = jax.lax.rsqrt(sqf - muf * muf + 1e-5)
    y = ((x2 - muf) * rstdf) * vec_ref[6:7, :] + vec_ref[7:8, :]

    eye = (jax.lax.broadcasted_iota(jnp.int32, (bs, bs), 0)
           == jax.lax.broadcasted_iota(jnp.int32, (bs, bs), 1))
    mf = jax.lax.dot_general(eye.astype(jnp.float32), m.astype(jnp.float32),
                             (((1,), (1,)), ((), ())),
                             preferred_element_type=jnp.float32)
    out_ref[...] = y * mf + a * (1.0 - mf)


def kernel(prompt_embeds, id_embeds, class_tokens_mask, ln1_g, ln1_b,
           w1_1, b1_1, w2_1, b2_1, ln2_g, ln2_b, w1_2, b1_2, w2_2, b2_2,
           lnf_g, lnf_b, *, interpret=False):
    B, S, D = prompt_embeds.shape
    bs = _BS
    nb = S // bs
    flat_prompt = prompt_embeds.reshape(S, D)
    flat_id = id_embeds.reshape(-1, D).astype(jnp.bfloat16)
    M = flat_id.shape[0]
    nid_b = M // bs

    mask = class_tokens_mask.reshape(S).astype(jnp.int32)
    mask3 = mask.reshape(nb, 1, bs)
    counts = jnp.sum(mask.reshape(nb, bs), axis=1)
    excl = jnp.concatenate([jnp.zeros((1,), jnp.int32),
                            jnp.cumsum(counts)[:-1].astype(jnp.int32)])
    b0 = jnp.minimum(excl // bs, nid_b - 1)
    b1 = jnp.minimum(b0 + 1, nid_b - 1)
    scal = jnp.stack([excl, b0, b1]).astype(jnp.int32)

    vec = jnp.stack([b1_1, b2_1, ln2_g, ln2_b, b1_2, b2_2, lnf_g, lnf_b])
    ln1 = jnp.stack([ln1_g, ln1_b])

    bf = jnp.bfloat16
    grid_spec = pltpu.PrefetchScalarGridSpec(
        num_scalar_prefetch=1,
        grid=(nb,),
        in_specs=[
            pl.BlockSpec((bs, D), lambda i, s: (i, 0)),
            pl.BlockSpec((bs, D), lambda i, s: (s[1, i], 0)),
            pl.BlockSpec((bs, D), lambda i, s: (s[2, i], 0)),
            pl.BlockSpec((1, 1, bs), lambda i, s: (i, 0, 0)),
            pl.BlockSpec((D, 2 * D), lambda i, s: (0, 0)),
            pl.BlockSpec((D, D), lambda i, s: (0, 0)),
            pl.BlockSpec((D, D), lambda i, s: (0, 0)),
            pl.BlockSpec((D, D), lambda i, s: (0, 0)),
            pl.BlockSpec((8, D), lambda i, s: (0, 0)),
            pl.BlockSpec((2, 2 * D), lambda i, s: (0, 0)),
        ],
        out_specs=pl.BlockSpec((bs, D), lambda i, s: (i, 0)),
        scratch_shapes=[
            pltpu.VMEM((D, D), bf), pltpu.VMEM((D, D), bf),
            pltpu.VMEM((D, D), bf), pltpu.VMEM((D, D), bf),
            pltpu.VMEM((D, D), bf),
        ],
    )
    out = pl.pallas_call(
        _fuse_body,
        grid_spec=grid_spec,
        out_shape=jax.ShapeDtypeStruct((S, D), jnp.float32),
        compiler_params=pltpu.CompilerParams(
            dimension_semantics=("arbitrary",),
            vmem_limit_bytes=100 * 1024 * 1024),
        interpret=interpret,
    )(scal, flat_prompt, flat_id, flat_id, mask3,
      w1_1, w2_1, w1_2, w2_2, vec, ln1)
    return out.reshape(B, S, D)


# speedup vs baseline: 1.2080x; 1.0008x over previous
import jax
import jax.numpy as jnp
from jax.experimental import pallas as pl
from jax.experimental.pallas import tpu as pltpu

_BS = 256


def _gelu_exact(x):
    return x * 0.5 * (1.0 + jax.lax.erf(x * 0.7071067811865476))


def _fuse_body(sref, prompt_ref, ida_ref, idb_ref, mask_ref,
               w11_ref, w21_ref, w12_ref, w22_ref,
               vec_ref, ln1_ref, out_ref,
               w11a_s, w11b_s, w21_s, w12_s, w22_s):
    i = pl.program_id(0)
    bs = _BS
    D = prompt_ref.shape[1]

    @pl.when(i == 0)
    def _cast_weights():
        w11a_s[...] = w11_ref[:, :D].astype(jnp.bfloat16)
        w11b_s[...] = w11_ref[:, D:].astype(jnp.bfloat16)
        w21_s[...] = w21_ref[...].astype(jnp.bfloat16)
        w12_s[...] = w12_ref[...].astype(jnp.bfloat16)
        w22_s[...] = w22_ref[...].astype(jnp.bfloat16)

    a = prompt_ref[...]
    m = mask_ref[0]
    tri = (jax.lax.broadcasted_iota(jnp.int32, (bs, bs), 0)
           <= jax.lax.broadcasted_iota(jnp.int32, (bs, bs), 1))
    cs_f = jnp.dot(m.astype(jnp.bfloat16), tri.astype(jnp.bfloat16),
                   preferred_element_type=jnp.float32)
    cs = cs_f.astype(jnp.int32)
    excl_i = sref[0, i]
    b0_i = sref[1, i]
    off = excl_i + cs - 1 - b0_i * bs
    off = jnp.clip(off, 0, 2 * bs - 1)

    iota = jax.lax.broadcasted_iota(jnp.int32, (2 * bs, bs), 0)
    ohT = (iota == off).astype(jnp.bfloat16)

    dimn = (((0,), (0,)), ((), ()))
    p = jax.lax.dot_general(ohT[:bs], ida_ref[...], dimn,
                            preferred_element_type=jnp.float32)
    p = p + jax.lax.dot_general(ohT[bs:], idb_ref[...], dimn,
                                preferred_element_type=jnp.float32)

    inv = 1.0 / (2 * D)
    mu = (jnp.sum(a, axis=1, keepdims=True)
          + jnp.sum(p, axis=1, keepdims=True)) * inv
    sq = (jnp.sum(a * a, axis=1, keepdims=True)
          + jnp.sum(p * p, axis=1, keepdims=True)) * inv
    rstd = jax.lax.rsqrt(sq - mu * mu + 1e-5)
    na = ((a - mu) * rstd) * ln1_ref[0:1, :D] + ln1_ref[1:2, :D]
    npair = ((p - mu) * rstd) * ln1_ref[0:1, D:] + ln1_ref[1:2, D:]

    def mm(x, w_s):
        return jax.lax.dot_general(
            x.astype(jnp.bfloat16), w_s[...], (((1,), (1,)), ((), ())),
            preferred_element_type=jnp.float32)

    h = mm(na, w11a_s) + mm(npair, w11b_s) + vec_ref[0:1, :]
    h = _gelu_exact(h)
    h = mm(h, w21_s) + vec_ref[1:2, :]
    x1 = h + a

    mu2 = jnp.mean(x1, axis=1, keepdims=True)
    sq2 = jnp.mean(x1 * x1, axis=1, keepdims=True)
    rstd2 = jax.lax.rsqrt(sq2 - mu2 * mu2 + 1e-5)
    n2 = ((x1 - mu2) * rstd2) * vec_ref[2:3, :] + vec_ref[3:4, :]
    h = mm(n2, w12_s) + vec_ref[4:5, :]
    h = _gelu_exact(h)
    h = mm(h, w22_s) + vec_ref[5:6, :]
    x2 = h + x1

    muf = jnp.mean(x2, axis=1, keepdims=True)
    sqf = jnp.mean(x2 * x2, axis=1, keepdims=True)
    rstdf = jax.lax.rsqrt(sqf - muf * muf + 1e-5)
    y = ((x2 - muf) * rstdf) * vec_ref[6:7, :] + vec_ref[7:8, :]

    eye = (jax.lax.broadcasted_iota(jnp.int32, (bs, bs), 0)
           == jax.lax.broadcasted_iota(jnp.int32, (bs, bs), 1))
    mf = jax.lax.dot_general(eye.astype(jnp.float32), m.astype(jnp.float32),
                             (((1,), (1,)), ((), ())),
                             preferred_element_type=jnp.float32)
    out_ref[...] = y * mf + a * (1.0 - mf)


def kernel(prompt_embeds, id_embeds, class_tokens_mask, ln1_g, ln1_b,
           w1_1, b1_1, w2_1, b2_1, ln2_g, ln2_b, w1_2, b1_2, w2_2, b2_2,
           lnf_g, lnf_b):
    B, S, D = prompt_embeds.shape
    bs = _BS
    nb = S // bs
    flat_prompt = prompt_embeds.reshape(S, D)
    flat_id = id_embeds.reshape(-1, D).astype(jnp.bfloat16)
    M = flat_id.shape[0]
    nid_b = M // bs

    mask = class_tokens_mask.reshape(S).astype(jnp.int32)
    mask3 = mask.reshape(nb, 1, bs)
    counts = jnp.sum(mask.reshape(nb, bs), axis=1)
    excl = jnp.concatenate([jnp.zeros((1,), jnp.int32),
                            jnp.cumsum(counts)[:-1].astype(jnp.int32)])
    b0 = jnp.minimum(excl // bs, nid_b - 1)
    b1 = jnp.minimum(b0 + 1, nid_b - 1)
    scal = jnp.stack([excl, b0, b1]).astype(jnp.int32)

    vec = jnp.stack([b1_1, b2_1, ln2_g, ln2_b, b1_2, b2_2, lnf_g, lnf_b])
    ln1 = jnp.stack([ln1_g, ln1_b])

    bf = jnp.bfloat16
    grid_spec = pltpu.PrefetchScalarGridSpec(
        num_scalar_prefetch=1,
        grid=(nb,),
        in_specs=[
            pl.BlockSpec((bs, D), lambda i, s: (i, 0)),
            pl.BlockSpec((bs, D), lambda i, s: (s[1, i], 0)),
            pl.BlockSpec((bs, D), lambda i, s: (s[2, i], 0)),
            pl.BlockSpec((1, 1, bs), lambda i, s: (i, 0, 0)),
            pl.BlockSpec((D, 2 * D), lambda i, s: (0, 0)),
            pl.BlockSpec((D, D), lambda i, s: (0, 0)),
            pl.BlockSpec((D, D), lambda i, s: (0, 0)),
            pl.BlockSpec((D, D), lambda i, s: (0, 0)),
            pl.BlockSpec((8, D), lambda i, s: (0, 0)),
            pl.BlockSpec((2, 2 * D), lambda i, s: (0, 0)),
        ],
        out_specs=pl.BlockSpec((bs, D), lambda i, s: (i, 0)),
        scratch_shapes=[
            pltpu.VMEM((D, D), bf), pltpu.VMEM((D, D), bf),
            pltpu.VMEM((D, D), bf), pltpu.VMEM((D, D), bf),
            pltpu.VMEM((D, D), bf),
        ],
    )
    out = pl.pallas_call(
        _fuse_body,
        grid_spec=grid_spec,
        out_shape=jax.ShapeDtypeStruct((S, D), jnp.float32),
        compiler_params=pltpu.CompilerParams(
            dimension_semantics=("arbitrary",),
            vmem_limit_bytes=100 * 1024 * 1024),
    )(scal, flat_prompt, flat_id, flat_id, mask3,
      w1_1, w2_1, w1_2, w2_2, vec, ln1)
    return out.reshape(B, S, D)
